# baseline (speedup 1.0000x reference)
import functools
from math import ceil

import numpy as np
import jax
import jax.numpy as jnp
from jax.experimental import pallas as pl
from jax.experimental.pallas import tpu as pltpu

_SEQ_LEN = 336
_D_MODEL = 512
_NUM_VARIATES = 11
_BATCH = 16


def _compute_patch_sizes(seq_len):
    freqs = np.fft.rfftfreq(seq_len)[1:]
    periods = 1.0 / freqs
    return np.unique(np.floor(periods).astype(np.int64))[::-1].copy()


_PATCH_SIZES = [int(p) for p in _compute_patch_sizes(_SEQ_LEN)]
_NS = [ceil(_SEQ_LEN / p) for p in _PATCH_SIZES]


def _sin_pe_np(L, d):
    pos = np.arange(L, dtype=np.float64)[:, None]
    div = np.exp(np.arange(0, d, 2, dtype=np.float64) * (-np.log(10000.0) / d))
    pe = np.zeros((L, d), dtype=np.float64)
    pe[:, 0::2] = np.sin(pos * div)
    pe[:, 1::2] = np.cos(pos * div)
    return pe


def _pe_rows_np(C, N, d_model):
    dh = d_model // 2
    pe = np.zeros((C, N, d_model), dtype=np.float32)
    pe[:, :, :dh] = _sin_pe_np(C, dh)[:, None, :]
    pe[:, :, dh:] = _sin_pe_np(N, d_model - dh)[None, :, :]
    return pe.reshape(C * N, d_model)


_PE_ROWS = {n: jnp.asarray(_pe_rows_np(_NUM_VARIATES, n, _D_MODEL))
            for n in sorted(set(_NS))}


def _expert_body(order_ref, x_ref, w_ref, pe_ref, o_ref):
    xi = x_ref[0].astype(jnp.bfloat16)
    w = w_ref[...].astype(jnp.bfloat16)
    acc = jax.lax.dot_general(
        xi, w, (((1,), (1,)), ((), ())),
        preferred_element_type=jnp.float32)
    o_ref[0] = acc + pe_ref[...]


@functools.partial(jax.jit, static_argnums=(0, 1))
def _expert_call(p, n, x_rows, w, pe, order):
    rows = _NUM_VARIATES * n
    grid_spec = pltpu.PrefetchScalarGridSpec(
        num_scalar_prefetch=1,
        grid=(_BATCH,),
        in_specs=[
            pl.BlockSpec((1, rows, p), lambda b, order: (order[b], 0, 0)),
            pl.BlockSpec((_D_MODEL, p), lambda b, order: (0, 0)),
            pl.BlockSpec((rows, _D_MODEL), lambda b, order: (0, 0)),
        ],
        out_specs=pl.BlockSpec((1, rows, _D_MODEL), lambda b, order: (b, 0, 0)),
    )
    out = pl.pallas_call(
        _expert_body,
        grid_spec=grid_spec,
        out_shape=jax.ShapeDtypeStruct((_BATCH, rows, _D_MODEL), jnp.float32),
    )(order, x_rows, w, pe)
    return out


def kernel(x, gates, Ws):
    batch_ar = jnp.arange(_BATCH, dtype=jnp.int32)[:, None]
    keys = jnp.where(gates != 0, jnp.int32(0), jnp.int32(1)) * (_BATCH + 1) + batch_ar
    orders = jnp.argsort(keys, axis=0, stable=True).astype(jnp.int32)
    orders = orders.T

    xt = jnp.swapaxes(x, 1, 2)
    xt_pad = jnp.concatenate(
        [xt, jnp.broadcast_to(xt[:, :, -1:], (_BATCH, _NUM_VARIATES, _SEQ_LEN))],
        axis=-1)

    outs = []
    for i, p in enumerate(_PATCH_SIZES):
        n = _NS[i]
        x_rows = xt_pad[:, :, : n * p].reshape(_BATCH, _NUM_VARIATES * n, p)
        out = _expert_call(p, n, x_rows, Ws[i], _PE_ROWS[n], orders[i])
        outs.append(out.reshape(_BATCH, _NUM_VARIATES, n, _D_MODEL))
    return tuple(outs)


# speedup vs baseline: 1.1256x; 1.1256x over previous
from math import ceil

import numpy as np
import jax
import jax.numpy as jnp
from jax.experimental import pallas as pl
from jax.experimental.pallas import tpu as pltpu

_SEQ_LEN = 336
_D_MODEL = 512
_C = 11
_B = 16


def _compute_patch_sizes(seq_len):
    freqs = np.fft.rfftfreq(seq_len)[1:]
    periods = 1.0 / freqs
    return np.unique(np.floor(periods).astype(np.int64))[::-1].copy()


_PATCH_SIZES = [int(p) for p in _compute_patch_sizes(_SEQ_LEN)]
_NS = [ceil(_SEQ_LEN / p) for p in _PATCH_SIZES]
_NE = len(_PATCH_SIZES)


def _sin_pe_np(L, d):
    pos = np.arange(L, dtype=np.float64)[:, None]
    div = np.exp(np.arange(0, d, 2, dtype=np.float64) * (-np.log(10000.0) / d))
    pe = np.zeros((L, d), dtype=np.float64)
    pe[:, 0::2] = np.sin(pos * div)
    pe[:, 1::2] = np.cos(pos * div)
    return pe


def _pe_rows_np(C, N, d_model):
    dh = d_model // 2
    pe = np.zeros((C, N, d_model), dtype=np.float32)
    pe[:, :, :dh] = _sin_pe_np(C, dh)[:, None, :]
    pe[:, :, dh:] = _sin_pe_np(N, d_model - dh)[None, :, :]
    return pe.reshape(C * N, d_model)


_PE_ROWS = {n: _pe_rows_np(_C, n, _D_MODEL) for n in sorted(set(_NS))}


def _fused_body(ord_ref, *refs):
    xs = refs[0:_NE]
    ws = refs[_NE:2 * _NE]
    pes = refs[2 * _NE:3 * _NE]
    outs = refs[3 * _NE:]
    for i in range(_NE):
        xi = xs[i][0, 0]
        acc = jax.lax.dot_general(
            xi, ws[i][...], (((1,), (1,)), ((), ())),
            preferred_element_type=jnp.float32)
        outs[i][0, 0] = acc + pes[i][0]


def kernel(x, gates, Ws):
    batch_ar = jnp.arange(_B, dtype=jnp.int32)[:, None]
    keys = jnp.where(gates != 0, jnp.int32(0), jnp.int32(1)) * (_B + 1) + batch_ar
    orders = jnp.argsort(keys, axis=0, stable=True).astype(jnp.int32).T

    xt = jnp.swapaxes(x, 1, 2).astype(jnp.bfloat16)
    xt_pad = jnp.concatenate(
        [xt, jnp.broadcast_to(xt[:, :, -1:], (_B, _C, _SEQ_LEN))], axis=-1)

    x_ops, w_ops, pe_ops = [], [], []
    x_specs, w_specs, pe_specs, out_specs, out_shapes = [], [], [], [], []
    for i, p in enumerate(_PATCH_SIZES):
        n = _NS[i]
        x_ops.append(xt_pad[:, :, : n * p].reshape(_B, _C, n, p))
        w_ops.append(Ws[i].astype(jnp.bfloat16))
        pe_ops.append(_PE_ROWS[n].reshape(_C, n, _D_MODEL))

        def _x_idx(c, b, ords, i=i):
            return (ords[i, b], c, 0, 0)

        x_specs.append(pl.BlockSpec((1, 1, n, p), _x_idx))
        w_specs.append(pl.BlockSpec((_D_MODEL, p), lambda c, b, ords: (0, 0)))
        pe_specs.append(pl.BlockSpec((1, n, _D_MODEL),
                                      lambda c, b, ords: (c, 0, 0)))
        out_specs.append(pl.BlockSpec((1, 1, n, _D_MODEL),
                                      lambda c, b, ords: (b, c, 0, 0)))
        out_shapes.append(
            jax.ShapeDtypeStruct((_B, _C, n, _D_MODEL), jnp.float32))

    grid_spec = pltpu.PrefetchScalarGridSpec(
        num_scalar_prefetch=1,
        grid=(_C, _B),
        in_specs=x_specs + w_specs + pe_specs,
        out_specs=out_specs,
    )
    outs = pl.pallas_call(
        _fused_body,
        grid_spec=grid_spec,
        out_shape=tuple(out_shapes),
    )(orders, *x_ops, *w_ops, *pe_ops)
    return tuple(outs)


# speedup vs baseline: 1.1275x; 1.0017x over previous
from math import ceil

import numpy as np
import jax
import jax.numpy as jnp
from jax.experimental import pallas as pl
from jax.experimental.pallas import tpu as pltpu

_SEQ_LEN = 336
_D_MODEL = 512
_C = 11
_B = 16


def _compute_patch_sizes(seq_len):
    freqs = np.fft.rfftfreq(seq_len)[1:]
    periods = 1.0 / freqs
    return np.unique(np.floor(periods).astype(np.int64))[::-1].copy()


_PATCH_SIZES = [int(p) for p in _compute_patch_sizes(_SEQ_LEN)]
_NS = [ceil(_SEQ_LEN / p) for p in _PATCH_SIZES]
_NE = len(_PATCH_SIZES)
_NS_PAD = [((n + 7) // 8) * 8 for n in _NS]
_N_TOTAL = sum(_NS_PAD)
_OFFS = np.concatenate([[0], np.cumsum(_NS_PAD)]).astype(np.int64)


def _sin_pe_np(L, d):
    pos = np.arange(L, dtype=np.float64)[:, None]
    div = np.exp(np.arange(0, d, 2, dtype=np.float64) * (-np.log(10000.0) / d))
    pe = np.zeros((L, d), dtype=np.float64)
    pe[:, 0::2] = np.sin(pos * div)
    pe[:, 1::2] = np.cos(pos * div)
    return pe


def _pe_full_np(C, N, d_model):
    dh = d_model // 2
    pe = np.zeros((C, N, d_model), dtype=np.float32)
    pe[:, :, :dh] = _sin_pe_np(C, dh)[:, None, :]
    pe[:, :, dh:] = _sin_pe_np(N, d_model - dh)[None, :, :]
    return pe


_PE_CAT = np.concatenate(
    [np.pad(_pe_full_np(_C, n, _D_MODEL), ((0, 0), (0, npad - n), (0, 0)))
     for n, npad in zip(_NS, _NS_PAD)], axis=1)


def _fused_body(ord_ref, *refs):
    xs = refs[0:_NE]
    ws = refs[_NE:2 * _NE]
    pe_ref = refs[2 * _NE]
    outs = refs[2 * _NE + 1:]
    b = pl.program_id(1)
    for i in range(_NE):
        src = ord_ref[i, b]
        xi = xs[i][src, 0]
        acc = jax.lax.dot_general(
            xi, ws[i][...], (((1,), (1,)), ((), ())),
            preferred_element_type=jnp.float32)
        lo = int(_OFFS[i])
        outs[i][0, 0] = acc + pe_ref[0, lo:lo + _NS[i]]


def kernel(x, gates, Ws):
    batch_ar = jnp.arange(_B, dtype=jnp.int32)[:, None]
    keys = jnp.where(gates != 0, jnp.int32(0), jnp.int32(1)) * (_B + 1) + batch_ar
    orders = jnp.argsort(keys, axis=0, stable=True).astype(jnp.int32).T

    xt = jnp.swapaxes(x, 1, 2).astype(jnp.bfloat16)
    xt_pad = jnp.concatenate(
        [xt, jnp.broadcast_to(xt[:, :, -1:], (_B, _C, _SEQ_LEN))], axis=-1)

    x_ops, w_ops = [], []
    x_specs, w_specs, out_specs, out_shapes = [], [], [], []
    for i, p in enumerate(_PATCH_SIZES):
        n = _NS[i]
        x_ops.append(xt_pad[:, :, : n * p].reshape(_B, _C, n, p))
        w_ops.append(Ws[i].astype(jnp.bfloat16))
        x_specs.append(
            pl.BlockSpec((_B, 1, n, p), lambda c, b, ords: (0, c, 0, 0)))
        w_specs.append(pl.BlockSpec((_D_MODEL, p), lambda c, b, ords: (0, 0)))
        out_specs.append(pl.BlockSpec((1, 1, n, _D_MODEL),
                                      lambda c, b, ords: (b, c, 0, 0)))
        out_shapes.append(
            jax.ShapeDtypeStruct((_B, _C, n, _D_MODEL), jnp.float32))

    pe_spec = pl.BlockSpec((1, _N_TOTAL, _D_MODEL),
                           lambda c, b, ords: (c, 0, 0))

    grid_spec = pltpu.PrefetchScalarGridSpec(
        num_scalar_prefetch=1,
        grid=(_C, _B),
        in_specs=x_specs + w_specs + [pe_spec],
        out_specs=out_specs,
    )
    outs = pl.pallas_call(
        _fused_body,
        grid_spec=grid_spec,
        out_shape=tuple(out_shapes),
    )(orders, *x_ops, *w_ops, _PE_CAT)
    return tuple(outs)


# speedup vs baseline: 1.2829x; 1.1379x over previous
from math import ceil

import numpy as np
import jax
import jax.numpy as jnp
from jax.experimental import pallas as pl
from jax.experimental.pallas import tpu as pltpu

_SEQ_LEN = 336
_DH = 256
_D_MODEL = 512
_C = 11
_B = 16


def _compute_patch_sizes(seq_len):
    freqs = np.fft.rfftfreq(seq_len)[1:]
    periods = 1.0 / freqs
    return np.unique(np.floor(periods).astype(np.int64))[::-1].copy()


_PATCH_SIZES = [int(p) for p in _compute_patch_sizes(_SEQ_LEN)]
_NS = [ceil(_SEQ_LEN / p) for p in _PATCH_SIZES]
_NE = len(_PATCH_SIZES)
_NPS = [n * p for n, p in zip(_NS, _PATCH_SIZES)]
_UNIQUE_NPS = sorted(set(_NPS))
_NP_MAX = max(_UNIQUE_NPS)
_NP_IDX = {npv: j for j, npv in enumerate(_UNIQUE_NPS)}

_NS_PAD = [((n + 7) // 8) * 8 for n in _NS]
_N_TOTAL = sum(_NS_PAD)
_OFFS = np.concatenate([[0], np.cumsum(_NS_PAD)]).astype(np.int64)
_NPS_PAD = [((v + 7) // 8) * 8 for v in _NPS]
_L_TOTAL = sum(_NPS_PAD)
_LOFFS = np.concatenate([[0], np.cumsum(_NPS_PAD)]).astype(np.int64)
_POFFS = np.concatenate([[0], np.cumsum(_PATCH_SIZES)]).astype(np.int64)


def _sin_pe_np(L, d):
    pos = np.arange(L, dtype=np.float64)[:, None]
    div = np.exp(np.arange(0, d, 2, dtype=np.float64) * (-np.log(10000.0) / d))
    pe = np.zeros((L, d), dtype=np.float64)
    pe[:, 0::2] = np.sin(pos * div)
    pe[:, 1::2] = np.cos(pos * div)
    return pe


_PE_C = _sin_pe_np(_C, _DH).astype(np.float32).reshape(_C, 1, _DH)
_PE_N = np.concatenate(
    [np.pad(_sin_pe_np(n, _D_MODEL - _DH).astype(np.float32),
            ((0, npad - n), (0, 0)))
     for n, npad in zip(_NS, _NS_PAD)], axis=0)

_G_CAT_F32 = np.zeros((_N_TOTAL, _NP_MAX), dtype=np.float32)
for _i in range(_NE):
    for _m in range(_NS[_i]):
        _G_CAT_F32[int(_OFFS[_i]) + _m,
                   _m * _PATCH_SIZES[_i]:(_m + 1) * _PATCH_SIZES[_i]] = 1.0
_WL_COLS = np.zeros((_L_TOTAL,), dtype=np.int32)
for _i in range(_NE):
    _lo = int(_LOFFS[_i])
    _l = np.arange(_NPS_PAD[_i])
    _WL_COLS[_lo:_lo + _NPS_PAD[_i]] = int(_POFFS[_i]) + (_l % _PATCH_SIZES[_i])


def _routing_orders(gates):
    batch_ar = jnp.arange(_B, dtype=jnp.int32)[:, None]
    keys = jnp.where(gates != 0, jnp.int32(0), jnp.int32(_B + 1)) + batch_ar
    return jnp.argsort(keys, axis=0, stable=True).astype(jnp.int32).T


def _prep_body(*refs):
    x_ref = refs[0]
    xp_outs = refs[1:]
    xt = x_ref[0]
    xpad = jnp.concatenate(
        [xt, jnp.broadcast_to(xt[:, _SEQ_LEN - 1:], (_C, _NP_MAX - _SEQ_LEN))],
        axis=1)
    xpad = xpad.astype(jnp.bfloat16)
    for j, npv in enumerate(_UNIQUE_NPS):
        xp_outs[j][0, :, 0] = xpad[:, :npv]


def _prep_call(xt):
    xp_shapes = [jax.ShapeDtypeStruct((_B, _C, 1, npv), jnp.bfloat16)
                 for npv in _UNIQUE_NPS]
    outs = pl.pallas_call(
        _prep_body,
        grid=(_B,),
        in_specs=[pl.BlockSpec((1, _C, _SEQ_LEN), lambda b: (b, 0, 0))],
        out_specs=[pl.BlockSpec((1, _C, 1, npv), lambda b: (b, 0, 0, 0))
                   for npv in _UNIQUE_NPS],
        out_shape=tuple(xp_shapes),
    )(xt)
    return outs


_BB = 4


def _fused_body(ord_ref, *refs):
    nu = len(_UNIQUE_NPS)
    xs = refs[0:nu]
    wl_ref = refs[nu]
    g_ref = refs[nu + 1]
    pec_ref = refs[nu + 2]
    pen_ref = refs[nu + 3]
    outs = refs[nu + 4:]
    b0 = pl.program_id(1) * _BB
    pc = pec_ref[0]
    for i in range(_NE):
        n = _NS[i]
        npv = _NPS[i]
        lo = int(_OFFS[i])
        llo = int(_LOFFS[i])
        pe = jnp.concatenate(
            [jnp.broadcast_to(pc, (n, _DH)), pen_ref[lo:lo + n]], axis=1)
        g = g_ref[lo:lo + n, :npv]
        wl = wl_ref[llo:llo + npv, :]
        xr = xs[_NP_IDX[npv]]
        for db in range(_BB):
            src = ord_ref[i, b0 + db]
            xrow = xr[src, 0, 0]
            gx = g * xrow[None, :]
            acc = jax.lax.dot_general(
                gx, wl, (((1,), (0,)), ((), ())),
                preferred_element_type=jnp.float32)
            outs[i][db, 0] = acc + pe


def kernel(x, gates, Ws):
    orders = _routing_orders(gates)

    xt = jnp.swapaxes(x, 1, 2)
    xpads = _prep_call(xt)

    wst = jnp.concatenate(Ws, axis=1).astype(jnp.bfloat16)
    wl_cat = jnp.take(wst.T, jnp.asarray(_WL_COLS), axis=0)
    g_cat = jnp.asarray(_G_CAT_F32).astype(jnp.bfloat16)

    x_specs = [pl.BlockSpec((_B, 1, 1, npv), lambda c, b, ords: (0, c, 0, 0))
               for npv in _UNIQUE_NPS]
    out_specs, out_shapes = [], []
    for i in range(_NE):
        n = _NS[i]
        out_specs.append(pl.BlockSpec((_BB, 1, n, _D_MODEL),
                                      lambda c, b, ords: (b, c, 0, 0)))
        out_shapes.append(
            jax.ShapeDtypeStruct((_B, _C, n, _D_MODEL), jnp.float32))

    wl_spec = pl.BlockSpec((_L_TOTAL, _D_MODEL), lambda c, b, ords: (0, 0))
    g_spec = pl.BlockSpec((_N_TOTAL, _NP_MAX), lambda c, b, ords: (0, 0))
    pec_spec = pl.BlockSpec((1, 1, _DH), lambda c, b, ords: (c, 0, 0))
    pen_spec = pl.BlockSpec((_N_TOTAL, _DH), lambda c, b, ords: (0, 0))

    grid_spec = pltpu.PrefetchScalarGridSpec(
        num_scalar_prefetch=1,
        grid=(_C, _B // _BB),
        in_specs=x_specs + [wl_spec, g_spec, pec_spec, pen_spec],
        out_specs=out_specs,
    )
    outs = pl.pallas_call(
        _fused_body,
        grid_spec=grid_spec,
        out_shape=tuple(out_shapes),
    )(orders, *xpads, wl_cat, g_cat, _PE_C, _PE_N)
    return tuple(outs)


# speedup vs baseline: 2.2645x; 1.7651x over previous
from math import ceil

import numpy as np
import functools

import jax
import jax.numpy as jnp
from jax.experimental import pallas as pl
from jax.experimental.pallas import tpu as pltpu
from jax.experimental.pallas import tpu_sc as plsc

_SEQ_LEN = 336
_DH = 256
_D_MODEL = 512
_C = 11
_B = 16


def _compute_patch_sizes(seq_len):
    freqs = np.fft.rfftfreq(seq_len)[1:]
    periods = 1.0 / freqs
    return np.unique(np.floor(periods).astype(np.int64))[::-1].copy()


_PATCH_SIZES = [int(p) for p in _compute_patch_sizes(_SEQ_LEN)]
_NS = [ceil(_SEQ_LEN / p) for p in _PATCH_SIZES]
_NE = len(_PATCH_SIZES)
_NPS = [n * p for n, p in zip(_NS, _PATCH_SIZES)]
_UNIQUE_NPS = sorted(set(_NPS))
_NP_MAX = max(_UNIQUE_NPS)
_NP_IDX = {npv: j for j, npv in enumerate(_UNIQUE_NPS)}

_NS_PAD = [((n + 7) // 8) * 8 for n in _NS]
_N_TOTAL = sum(_NS_PAD)
_OFFS = np.concatenate([[0], np.cumsum(_NS_PAD)]).astype(np.int64)
_NPS_PAD = [((v + 7) // 8) * 8 for v in _NPS]
_L_TOTAL = sum(_NPS_PAD)
_LOFFS = np.concatenate([[0], np.cumsum(_NPS_PAD)]).astype(np.int64)
_POFFS = np.concatenate([[0], np.cumsum(_PATCH_SIZES)]).astype(np.int64)


def _sin_pe_np(L, d):
    pos = np.arange(L, dtype=np.float64)[:, None]
    div = np.exp(np.arange(0, d, 2, dtype=np.float64) * (-np.log(10000.0) / d))
    pe = np.zeros((L, d), dtype=np.float64)
    pe[:, 0::2] = np.sin(pos * div)
    pe[:, 1::2] = np.cos(pos * div)
    return pe


_PE_C = _sin_pe_np(_C, _DH).astype(np.float32).reshape(_C, 1, _DH)
_PE_N = np.concatenate(
    [np.pad(_sin_pe_np(n, _D_MODEL - _DH).astype(np.float32),
            ((0, npad - n), (0, 0)))
     for n, npad in zip(_NS, _NS_PAD)], axis=0)

_G_CAT_F32 = np.zeros((_N_TOTAL, _NP_MAX), dtype=np.float32)
for _i in range(_NE):
    for _m in range(_NS[_i]):
        _G_CAT_F32[int(_OFFS[_i]) + _m,
                   _m * _PATCH_SIZES[_i]:(_m + 1) * _PATCH_SIZES[_i]] = 1.0

_BB = 8
_R_CONST = np.zeros((_BB * max(_NS), _BB), dtype=np.float32)
for _r in range(_R_CONST.shape[0]):
    _R_CONST[_r, _r % _BB] = 1.0

_GROUPS = [(0, 24), (24, 30), (30, 32)]
_WL_COLS = np.zeros((_L_TOTAL,), dtype=np.int32)
for _i in range(_NE):
    _lo = int(_LOFFS[_i])
    _l = np.arange(_NPS_PAD[_i])
    _WL_COLS[_lo:_lo + _NPS_PAD[_i]] = int(_POFFS[_i]) + (_l % _PATCH_SIZES[_i])


def _routing_body(gt_hbm, out_hbm, gs, os_):
    cid = jax.lax.axis_index("c")

    @pl.when(cid == 0)
    def _():
        pltpu.sync_copy(gt_hbm, gs)

        def do_row(i, carry):
            def count_nz(b, cnz):
                nzb = gs[i, b] != 0.0

                @pl.when(nzb)
                def _():
                    os_[i, cnz] = b

                return cnz + jnp.where(nzb, jnp.int32(1), jnp.int32(0))

            tot_nz = jax.lax.fori_loop(0, _B, count_nz, jnp.int32(0))

            def place_z(b, cz):
                zb = gs[i, b] == 0.0

                @pl.when(zb)
                def _():
                    os_[i, tot_nz + cz] = b

                return cz + jnp.where(zb, jnp.int32(1), jnp.int32(0))

            jax.lax.fori_loop(0, _B, place_z, jnp.int32(0))
            return carry

        jax.lax.fori_loop(0, _NE, do_row, jnp.int32(0))
        pltpu.sync_copy(os_, out_hbm)


def _routing_orders(gates):
    scs_mesh = plsc.ScalarSubcoreMesh(axis_name="c", num_cores=2)
    routing = functools.partial(
        pl.kernel,
        out_type=jax.ShapeDtypeStruct((_NE, _B), jnp.int32),
        mesh=scs_mesh,
        scratch_types=[pltpu.SMEM((_NE, _B), jnp.float32),
                       pltpu.SMEM((_NE, _B), jnp.int32)],
    )(_routing_body)
    return routing(gates.T)


def _prep_body(*refs):
    x_ref = refs[0]
    xp_outs = refs[1:]
    xt = x_ref[0]
    xpad = jnp.concatenate(
        [xt, jnp.broadcast_to(xt[:, _SEQ_LEN - 1:], (_C, _NP_MAX - _SEQ_LEN))],
        axis=1)
    xpad = xpad.astype(jnp.bfloat16)
    for j, npv in enumerate(_UNIQUE_NPS):
        xp_outs[j][0, :, 0] = xpad[:, :npv]


def _prep_call(xt):
    xp_shapes = [jax.ShapeDtypeStruct((_B, _C, 1, npv), jnp.bfloat16)
                 for npv in _UNIQUE_NPS]
    outs = pl.pallas_call(
        _prep_body,
        grid=(_B,),
        in_specs=[pl.BlockSpec((1, _C, _SEQ_LEN), lambda b: (b, 0, 0))],
        out_specs=[pl.BlockSpec((1, _C, 1, npv), lambda b: (b, 0, 0, 0))
                   for npv in _UNIQUE_NPS],
        out_shape=tuple(xp_shapes),
    )(xt)
    return outs


def _make_fused_body(elo, ehi):
    nbase = int(_OFFS[elo])
    lbase = int(_LOFFS[elo])

    def _fused_body(ord_ref, *refs):
        nu = len(_UNIQUE_NPS)
        xs = refs[0:nu]
        wl_ref = refs[nu]
        g_ref = refs[nu + 1]
        r_ref = refs[nu + 2]
        pec_ref = refs[nu + 3]
        pen_ref = refs[nu + 4]
        outs = refs[nu + 5:]
        b0 = pl.program_id(1) * _BB
        pc = pec_ref[0]
        for oi, i in enumerate(range(elo, ehi)):
            n = _NS[i]
            npv = _NPS[i]
            lo = int(_OFFS[i]) - nbase
            llo = int(_LOFFS[i]) - lbase
            xr = xs[_NP_IDX[npv]]
            rows = [xr[ord_ref[i, b0 + db], 0, 0][None, :]
                    for db in range(_BB)]
            xstk = jnp.concatenate(rows, axis=0)
            xrep = jax.lax.dot_general(
                r_ref[:_BB * n, :], xstk, (((1,), (0,)), ((), ())),
                preferred_element_type=jnp.float32
            ).astype(jnp.bfloat16)
            g = g_ref[lo:lo + n, :npv]
            grep = jnp.broadcast_to(
                g[:, None, :], (n, _BB, npv)).reshape(_BB * n, npv)
            gx = grep * xrep
            acc = jax.lax.dot_general(
                gx, wl_ref[llo:llo + npv, :], (((1,), (0,)), ((), ())),
                preferred_element_type=jnp.float32)
            pen = pen_ref[lo:lo + n]
            pen_rep = jnp.broadcast_to(
                pen[:, None, :], (n, _BB, _DH)).reshape(_BB * n, _DH)
            pe = jnp.concatenate(
                [jnp.broadcast_to(pc, (_BB * n, _DH)), pen_rep], axis=1)
            outs[oi][0] = (acc + pe).reshape(n, _BB, _D_MODEL)

    return _fused_body


def _make_rowmajor_body(elo, ehi):
    nbase = int(_OFFS[elo])
    lbase = int(_LOFFS[elo])

    def _body(ord_ref, *refs):
        nu = len(_UNIQUE_NPS)
        xs = refs[0:nu]
        wl_ref = refs[nu]
        g_ref = refs[nu + 1]
        pec_ref = refs[nu + 2]
        pen_ref = refs[nu + 3]
        outs = refs[nu + 4:]
        b0 = pl.program_id(1) * _BB
        pc = pec_ref[0]
        for oi, i in enumerate(range(elo, ehi)):
            n = _NS[i]
            npv = _NPS[i]
            lo = int(_OFFS[i]) - nbase
            llo = int(_LOFFS[i]) - lbase
            xr = xs[_NP_IDX[npv]]
            g = g_ref[lo:lo + n, :npv]
            wl = wl_ref[llo:llo + npv, :]
            pe = jnp.concatenate(
                [jnp.broadcast_to(pc, (n, _DH)), pen_ref[lo:lo + n]], axis=1)
            for db in range(_BB):
                src = ord_ref[i, b0 + db]
                xrow = xr[src, 0, 0]
                gx = g * xrow[None, :]
                acc = jax.lax.dot_general(
                    gx, wl, (((1,), (0,)), ((), ())),
                    preferred_element_type=jnp.float32)
                outs[oi][db, 0] = acc + pe

    return _body


def kernel(x, gates, Ws):
    orders = _routing_orders(gates)

    xt = jnp.swapaxes(x, 1, 2)
    xpads = _prep_call(xt)

    wst = jnp.concatenate(Ws, axis=1).astype(jnp.bfloat16)
    wl_cat = jnp.take(wst.T, jnp.asarray(_WL_COLS), axis=0)

    x_specs = [pl.BlockSpec((_B, 1, 1, npv), lambda c, b, ords: (0, c, 0, 0))
               for npv in _UNIQUE_NPS]
    r_const = jnp.asarray(_R_CONST).astype(jnp.bfloat16)
    g_cat = jnp.asarray(_G_CAT_F32).astype(jnp.bfloat16)
    all_outs = []
    for elo, ehi in _GROUPS:
        nbase, nhi = int(_OFFS[elo]), int(_OFFS[ehi])
        lbase, lhi = int(_LOFFS[elo]), int(_LOFFS[ehi])
        out_specs, out_shapes = [], []
        for i in range(elo, ehi):
            n = _NS[i]
            out_specs.append(pl.BlockSpec((1, n, _BB, _D_MODEL),
                                          lambda c, b, ords: (c, 0, b, 0)))
            out_shapes.append(
                jax.ShapeDtypeStruct((_C, n, _B, _D_MODEL), jnp.float32))

        nrows = nhi - nbase
        lrows = lhi - lbase
        wl_spec = pl.BlockSpec((lrows, _D_MODEL), lambda c, b, ords: (0, 0))
        g_spec = pl.BlockSpec((nrows, _NP_MAX), lambda c, b, ords: (0, 0))
        r_spec = pl.BlockSpec(_R_CONST.shape, lambda c, b, ords: (0, 0))
        pec_spec = pl.BlockSpec((1, 1, _DH), lambda c, b, ords: (c, 0, 0))
        pen_spec = pl.BlockSpec((nrows, _DH), lambda c, b, ords: (0, 0))

        grid_spec = pltpu.PrefetchScalarGridSpec(
            num_scalar_prefetch=1,
            grid=(_C, _B // _BB),
            in_specs=x_specs + [wl_spec, g_spec, r_spec, pec_spec, pen_spec],
            out_specs=out_specs,
        )
        outs = pl.pallas_call(
            _make_fused_body(elo, ehi),
            grid_spec=grid_spec,
            out_shape=tuple(out_shapes),
        )(orders, *xpads, wl_cat[lbase:lhi], g_cat[nbase:nhi],
          r_const, _PE_C, _PE_N[nbase:nhi])
        all_outs.extend(outs)
    all_outs = [jnp.transpose(o, (2, 0, 1, 3)) for o in all_outs]

    elo, ehi = _GROUPS[-1][1], _NE
    nbase, nhi = int(_OFFS[elo]), int(_OFFS[ehi])
    lbase, lhi = int(_LOFFS[elo]), int(_LOFFS[ehi])
    out_specs = [pl.BlockSpec((_BB, 1, _NS[i], _D_MODEL),
                              lambda c, b, ords: (b, c, 0, 0))
                 for i in range(elo, ehi)]
    out_shapes = [jax.ShapeDtypeStruct((_B, _C, _NS[i], _D_MODEL), jnp.float32)
                  for i in range(elo, ehi)]
    grid_spec = pltpu.PrefetchScalarGridSpec(
        num_scalar_prefetch=1,
        grid=(_C, _B // _BB),
        in_specs=x_specs + [
            pl.BlockSpec((lhi - lbase, _D_MODEL), lambda c, b, ords: (0, 0)),
            pl.BlockSpec((nhi - nbase, _NP_MAX), lambda c, b, ords: (0, 0)),
            pl.BlockSpec((1, 1, _DH), lambda c, b, ords: (c, 0, 0)),
            pl.BlockSpec((nhi - nbase, _DH), lambda c, b, ords: (0, 0))],
        out_specs=out_specs,
    )
    tail = pl.pallas_call(
        _make_rowmajor_body(elo, ehi),
        grid_spec=grid_spec,
        out_shape=tuple(out_shapes),
    )(orders, *xpads, wl_cat[lbase:lhi], g_cat[nbase:nhi], _PE_C,
      _PE_N[nbase:nhi])
    all_outs.extend(tail)
    return tuple(all_outs)


# speedup vs baseline: 2.2916x; 1.0120x over previous
from math import ceil

import numpy as np
import functools

import jax
import jax.numpy as jnp
from jax.experimental import pallas as pl
from jax.experimental.pallas import tpu as pltpu
from jax.experimental.pallas import tpu_sc as plsc

_SEQ_LEN = 336
_DH = 256
_D_MODEL = 512
_C = 11
_B = 16


def _compute_patch_sizes(seq_len):
    freqs = np.fft.rfftfreq(seq_len)[1:]
    periods = 1.0 / freqs
    return np.unique(np.floor(periods).astype(np.int64))[::-1].copy()


_PATCH_SIZES = [int(p) for p in _compute_patch_sizes(_SEQ_LEN)]
_NS = [ceil(_SEQ_LEN / p) for p in _PATCH_SIZES]
_NE = len(_PATCH_SIZES)
_NPS = [n * p for n, p in zip(_NS, _PATCH_SIZES)]
_UNIQUE_NPS = sorted(set(_NPS))
_NP_MAX = max(_UNIQUE_NPS)
_NP_IDX = {npv: j for j, npv in enumerate(_UNIQUE_NPS)}

_NS_PAD = [((n + 7) // 8) * 8 for n in _NS]
_N_TOTAL = sum(_NS_PAD)
_OFFS = np.concatenate([[0], np.cumsum(_NS_PAD)]).astype(np.int64)
_NPS_PAD = [((v + 7) // 8) * 8 for v in _NPS]
_L_TOTAL = sum(_NPS_PAD)
_LOFFS = np.concatenate([[0], np.cumsum(_NPS_PAD)]).astype(np.int64)
_POFFS = np.concatenate([[0], np.cumsum(_PATCH_SIZES)]).astype(np.int64)


def _sin_pe_np(L, d):
    pos = np.arange(L, dtype=np.float64)[:, None]
    div = np.exp(np.arange(0, d, 2, dtype=np.float64) * (-np.log(10000.0) / d))
    pe = np.zeros((L, d), dtype=np.float64)
    pe[:, 0::2] = np.sin(pos * div)
    pe[:, 1::2] = np.cos(pos * div)
    return pe


_PE_C = _sin_pe_np(_C, _DH).astype(np.float32).reshape(_C, 1, _DH)
_PE_N = np.concatenate(
    [np.pad(_sin_pe_np(n, _D_MODEL - _DH).astype(np.float32),
            ((0, npad - n), (0, 0)))
     for n, npad in zip(_NS, _NS_PAD)], axis=0)

_G_CAT_F32 = np.zeros((_N_TOTAL, _NP_MAX), dtype=np.float32)
for _i in range(_NE):
    for _m in range(_NS[_i]):
        _G_CAT_F32[int(_OFFS[_i]) + _m,
                   _m * _PATCH_SIZES[_i]:(_m + 1) * _PATCH_SIZES[_i]] = 1.0

_BB = 8
_R_CONST = np.zeros((_BB * max(_NS), _BB), dtype=np.float32)
for _r in range(_R_CONST.shape[0]):
    _R_CONST[_r, _r % _BB] = 1.0

_GROUPS = [(0, 24), (24, 32)]
_WL_COLS = np.zeros((_L_TOTAL,), dtype=np.int32)
for _i in range(_NE):
    _lo = int(_LOFFS[_i])
    _l = np.arange(_NPS_PAD[_i])
    _WL_COLS[_lo:_lo + _NPS_PAD[_i]] = int(_POFFS[_i]) + (_l % _PATCH_SIZES[_i])


def _routing_body(gt_hbm, out_hbm, gs, os_):
    cid = jax.lax.axis_index("c")

    @pl.when(cid == 0)
    def _():
        pltpu.sync_copy(gt_hbm, gs)

        def do_row(i, carry):
            def count_nz(b, cnz):
                nzb = gs[i, b] != 0.0

                @pl.when(nzb)
                def _():
                    os_[i, cnz] = b

                return cnz + jnp.where(nzb, jnp.int32(1), jnp.int32(0))

            tot_nz = jax.lax.fori_loop(0, _B, count_nz, jnp.int32(0))

            def place_z(b, cz):
                zb = gs[i, b] == 0.0

                @pl.when(zb)
                def _():
                    os_[i, tot_nz + cz] = b

                return cz + jnp.where(zb, jnp.int32(1), jnp.int32(0))

            jax.lax.fori_loop(0, _B, place_z, jnp.int32(0))
            return carry

        jax.lax.fori_loop(0, _NE, do_row, jnp.int32(0))
        pltpu.sync_copy(os_, out_hbm)


def _routing_orders(gates):
    scs_mesh = plsc.ScalarSubcoreMesh(axis_name="c", num_cores=2)
    routing = functools.partial(
        pl.kernel,
        out_type=jax.ShapeDtypeStruct((_NE, _B), jnp.int32),
        mesh=scs_mesh,
        scratch_types=[pltpu.SMEM((_NE, _B), jnp.float32),
                       pltpu.SMEM((_NE, _B), jnp.int32)],
    )(_routing_body)
    return routing(gates.T)


def _prep_body(*refs):
    x_ref = refs[0]
    xp_outs = refs[1:]
    xt = x_ref[0]
    xpad = jnp.concatenate(
        [xt, jnp.broadcast_to(xt[:, _SEQ_LEN - 1:], (_C, _NP_MAX - _SEQ_LEN))],
        axis=1)
    xpad = xpad.astype(jnp.bfloat16)
    for j, npv in enumerate(_UNIQUE_NPS):
        xp_outs[j][0, :, 0] = xpad[:, :npv]


def _prep_call(xt):
    xp_shapes = [jax.ShapeDtypeStruct((_B, _C, 1, npv), jnp.bfloat16)
                 for npv in _UNIQUE_NPS]
    outs = pl.pallas_call(
        _prep_body,
        grid=(_B,),
        in_specs=[pl.BlockSpec((1, _C, _SEQ_LEN), lambda b: (b, 0, 0))],
        out_specs=[pl.BlockSpec((1, _C, 1, npv), lambda b: (b, 0, 0, 0))
                   for npv in _UNIQUE_NPS],
        out_shape=tuple(xp_shapes),
    )(xt)
    return outs


def _make_fused_body(elo, ehi):
    nbase = int(_OFFS[elo])
    lbase = int(_LOFFS[elo])

    def _fused_body(ord_ref, *refs):
        nu = len(_UNIQUE_NPS)
        xs = refs[0:nu]
        wl_ref = refs[nu]
        g_ref = refs[nu + 1]
        r_ref = refs[nu + 2]
        pec_ref = refs[nu + 3]
        pen_ref = refs[nu + 4]
        outs = refs[nu + 5:]
        b0 = pl.program_id(1) * _BB
        pc = pec_ref[0]
        for oi, i in enumerate(range(elo, ehi)):
            n = _NS[i]
            npv = _NPS[i]
            lo = int(_OFFS[i]) - nbase
            llo = int(_LOFFS[i]) - lbase
            xr = xs[_NP_IDX[npv]]
            rows = [xr[ord_ref[i, b0 + db], 0, 0][None, :]
                    for db in range(_BB)]
            xstk = jnp.concatenate(rows, axis=0)
            xrep = jax.lax.dot_general(
                r_ref[:_BB * n, :], xstk, (((1,), (0,)), ((), ())),
                preferred_element_type=jnp.float32
            ).astype(jnp.bfloat16)
            g = g_ref[lo:lo + n, :npv]
            grep = jnp.broadcast_to(
                g[:, None, :], (n, _BB, npv)).reshape(_BB * n, npv)
            gx = grep * xrep
            acc = jax.lax.dot_general(
                gx, wl_ref[llo:llo + npv, :], (((1,), (0,)), ((), ())),
                preferred_element_type=jnp.float32)
            pen = pen_ref[lo:lo + n]
            pen_rep = jnp.broadcast_to(
                pen[:, None, :], (n, _BB, _DH)).reshape(_BB * n, _DH)
            pe = jnp.concatenate(
                [jnp.broadcast_to(pc, (_BB * n, _DH)), pen_rep], axis=1)
            outs[oi][0] = (acc + pe).reshape(n, _BB, _D_MODEL)

    return _fused_body


def _make_rowmajor_body(elo, ehi):
    nbase = int(_OFFS[elo])
    lbase = int(_LOFFS[elo])

    def _body(ord_ref, *refs):
        nu = len(_UNIQUE_NPS)
        xs = refs[0:nu]
        wl_ref = refs[nu]
        g_ref = refs[nu + 1]
        pec_ref = refs[nu + 2]
        pen_ref = refs[nu + 3]
        outs = refs[nu + 4:]
        b0 = pl.program_id(1) * _BB
        pc = pec_ref[0]
        for oi, i in enumerate(range(elo, ehi)):
            n = _NS[i]
            npv = _NPS[i]
            lo = int(_OFFS[i]) - nbase
            llo = int(_LOFFS[i]) - lbase
            xr = xs[_NP_IDX[npv]]
            g = g_ref[lo:lo + n, :npv]
            wl = wl_ref[llo:llo + npv, :]
            pe = jnp.concatenate(
                [jnp.broadcast_to(pc, (n, _DH)), pen_ref[lo:lo + n]], axis=1)
            for db in range(_BB):
                src = ord_ref[i, b0 + db]
                xrow = xr[src, 0, 0]
                gx = g * xrow[None, :]
                acc = jax.lax.dot_general(
                    gx, wl, (((1,), (0,)), ((), ())),
                    preferred_element_type=jnp.float32)
                outs[oi][db, 0] = acc + pe

    return _body


def kernel(x, gates, Ws):
    orders = _routing_orders(gates)

    xt = jnp.swapaxes(x, 1, 2)
    xpads = _prep_call(xt)

    wst = jnp.concatenate(Ws, axis=1).astype(jnp.bfloat16)
    wl_cat = jnp.take(wst.T, jnp.asarray(_WL_COLS), axis=0)

    x_specs = [pl.BlockSpec((_B, 1, 1, npv), lambda c, b, ords: (0, c, 0, 0))
               for npv in _UNIQUE_NPS]
    r_const = jnp.asarray(_R_CONST).astype(jnp.bfloat16)
    g_cat = jnp.asarray(_G_CAT_F32).astype(jnp.bfloat16)
    all_outs = []
    for elo, ehi in _GROUPS:
        nbase, nhi = int(_OFFS[elo]), int(_OFFS[ehi])
        lbase, lhi = int(_LOFFS[elo]), int(_LOFFS[ehi])
        out_specs, out_shapes = [], []
        for i in range(elo, ehi):
            n = _NS[i]
            out_specs.append(pl.BlockSpec((1, n, _BB, _D_MODEL),
                                          lambda c, b, ords: (c, 0, b, 0)))
            out_shapes.append(
                jax.ShapeDtypeStruct((_C, n, _B, _D_MODEL), jnp.float32))

        nrows = nhi - nbase
        lrows = lhi - lbase
        wl_spec = pl.BlockSpec((lrows, _D_MODEL), lambda c, b, ords: (0, 0))
        g_spec = pl.BlockSpec((nrows, _NP_MAX), lambda c, b, ords: (0, 0))
        r_spec = pl.BlockSpec(_R_CONST.shape, lambda c, b, ords: (0, 0))
        pec_spec = pl.BlockSpec((1, 1, _DH), lambda c, b, ords: (c, 0, 0))
        pen_spec = pl.BlockSpec((nrows, _DH), lambda c, b, ords: (0, 0))

        grid_spec = pltpu.PrefetchScalarGridSpec(
            num_scalar_prefetch=1,
            grid=(_C, _B // _BB),
            in_specs=x_specs + [wl_spec, g_spec, r_spec, pec_spec, pen_spec],
            out_specs=out_specs,
        )
        outs = pl.pallas_call(
            _make_fused_body(elo, ehi),
            grid_spec=grid_spec,
            out_shape=tuple(out_shapes),
        )(orders, *xpads, wl_cat[lbase:lhi], g_cat[nbase:nhi],
          r_const, _PE_C, _PE_N[nbase:nhi])
        all_outs.extend(outs)
    all_outs = [jnp.transpose(o, (2, 0, 1, 3)) for o in all_outs]

    elo, ehi = _GROUPS[-1][1], _NE
    nbase, nhi = int(_OFFS[elo]), int(_OFFS[ehi])
    lbase, lhi = int(_LOFFS[elo]), int(_LOFFS[ehi])
    out_specs = [pl.BlockSpec((_BB, 1, _NS[i], _D_MODEL),
                              lambda c, b, ords: (b, c, 0, 0))
                 for i in range(elo, ehi)]
    out_shapes = [jax.ShapeDtypeStruct((_B, _C, _NS[i], _D_MODEL), jnp.float32)
                  for i in range(elo, ehi)]
    grid_spec = pltpu.PrefetchScalarGridSpec(
        num_scalar_prefetch=1,
        grid=(_C, _B // _BB),
        in_specs=x_specs + [
            pl.BlockSpec((lhi - lbase, _D_MODEL), lambda c, b, ords: (0, 0)),
            pl.BlockSpec((nhi - nbase, _NP_MAX), lambda c, b, ords: (0, 0)),
            pl.BlockSpec((1, 1, _DH), lambda c, b, ords: (c, 0, 0)),
            pl.BlockSpec((nhi - nbase, _DH), lambda c, b, ords: (0, 0))],
        out_specs=out_specs,
    )
    tail = pl.pallas_call(
        _make_rowmajor_body(elo, ehi),
        grid_spec=grid_spec,
        out_shape=tuple(out_shapes),
    )(orders, *xpads, wl_cat[lbase:lhi], g_cat[nbase:nhi], _PE_C,
      _PE_N[nbase:nhi])
    all_outs.extend(tail)
    return tuple(all_outs)
